# baseline (speedup 1.0000x reference)
import functools

import jax
import jax.numpy as jnp
from jax import lax
from jax.experimental import pallas as pl
from jax.experimental.pallas import tpu as pltpu
from jax.experimental.pallas import tpu_sc as plsc

N_NODES = 10000
N_EDGES = 320000
D = 128

NC, NS = 2, 16
NW = NC * NS
NPAD = 10240
CHUNK = 128
NCHUNK = 2560
CPW = NCHUNK // NW
TROWS = NPAD // NS

_mesh = plsc.VectorSubcoreMesh(
    core_axis_name="c", subcore_axis_name="s", num_cores=NC, num_subcores=NS
)


HALF = NPAD // 2
SEG = HALF // NS


@functools.partial(
    pl.kernel,
    mesh=_mesh,
    out_type=jax.ShapeDtypeStruct((NC, NPAD, D), jnp.float32),
    scratch_types=[
        pltpu.VMEM((CPW, CHUNK), jnp.int32),
        pltpu.VMEM((CHUNK, D), jnp.float32),
        pltpu.VMEM_SHARED((NPAD, D), jnp.float32),
        pltpu.SemaphoreType.DMA,
    ],
)
def _deg_kernel(dst_hbm, ones_hbm, zeros_hbm, out_hbm, dst_v, ones_v, acc, sem):
    c = lax.axis_index("c")
    s = lax.axis_index("s")
    w = s * NC + c
    pltpu.sync_copy(dst_hbm.at[pl.ds(w * CPW, CPW)], dst_v)
    pltpu.sync_copy(ones_hbm, ones_v)
    pltpu.sync_copy(zeros_hbm, acc.at[pl.ds(s * TROWS, TROWS)])
    plsc.subcore_barrier()

    for k in range(3):
        pltpu.async_copy(ones_v, acc.at[dst_v.at[k]], sem, add=True)

    def body(j, carry):
        @pl.when(j < CPW - 3)
        def _():
            pltpu.async_copy(ones_v, acc.at[dst_v.at[j + 3]], sem, add=True)

        pltpu.make_async_copy(ones_v, acc.at[dst_v.at[j]], sem).wait()
        return carry

    lax.fori_loop(0, CPW, body, 0)
    plsc.subcore_barrier()
    pltpu.sync_copy(
        acc.at[pl.ds(s * TROWS, TROWS)], out_hbm.at[c, pl.ds(s * TROWS, TROWS)]
    )


@functools.partial(
    pl.kernel,
    mesh=_mesh,
    out_type=jax.ShapeDtypeStruct((NC, NPAD, D), jnp.float32),
    scratch_types=[
        pltpu.VMEM((CPW // 2, CHUNK), jnp.int32),
        pltpu.VMEM((CPW // 2, CHUNK), jnp.int32),
        pltpu.VMEM((CHUNK, D), jnp.float32),
        pltpu.VMEM((CHUNK, D), jnp.float32),
        pltpu.VMEM_SHARED((NPAD, D), jnp.float32),
        pltpu.SemaphoreType.DMA,
        pltpu.SemaphoreType.DMA,
        pltpu.SemaphoreType.DMA,
        pltpu.SemaphoreType.DMA,
    ],
)
def _mp_kernel(h_hbm, src_hbm, dst_hbm, zeros_hbm, out_hbm,
               src_v, dst_v, g0, g1, acc, sg0, sg1, ss0, ss1):
    c = lax.axis_index("c")
    s = lax.axis_index("s")
    w = s * NC + c
    pltpu.sync_copy(zeros_hbm, acc.at[pl.ds(s * TROWS, TROWS)])
    plsc.subcore_barrier()

    half = CPW // 2
    npair = half // 2
    for h_i in range(2):
        base = w * CPW + h_i * half
        pltpu.sync_copy(src_hbm.at[pl.ds(base, half)], src_v)
        pltpu.sync_copy(dst_hbm.at[pl.ds(base, half)], dst_v)
        pltpu.async_copy(h_hbm.at[src_v.at[0]], g0, sg0)

        def body(g, carry):
            j0 = 2 * g
            j1 = j0 + 1
            pltpu.async_copy(h_hbm.at[src_v.at[j1]], g1, sg1)
            pltpu.make_async_copy(h_hbm.at[src_v.at[j0]], g0, sg0).wait()
            pltpu.async_copy(g0, acc.at[dst_v.at[j0]], ss0, add=True)
            pltpu.make_async_copy(h_hbm.at[src_v.at[j1]], g1, sg1).wait()
            pltpu.async_copy(g1, acc.at[dst_v.at[j1]], ss1, add=True)
            pltpu.make_async_copy(g0, acc.at[dst_v.at[j0]], ss0).wait()

            @pl.when(g < npair - 1)
            def _():
                pltpu.async_copy(h_hbm.at[src_v.at[j0 + 2]], g0, sg0)

            pltpu.make_async_copy(g1, acc.at[dst_v.at[j1]], ss1).wait()
            return carry

        lax.fori_loop(0, npair, body, 0)
    plsc.subcore_barrier()
    pltpu.sync_copy(
        acc.at[pl.ds(s * TROWS, TROWS)], out_hbm.at[c, pl.ds(s * TROWS, TROWS)]
    )


BM = 1280


def _k1a_body(x_ref, w_ref, h_ref):
    h_ref[...] = jnp.dot(x_ref[...], w_ref[...],
                         preferred_element_type=jnp.float32)


def _k1_body(h_ref, degp_ref, h1s_ref, dis_ref):
    deg = degp_ref[0][:, 0:1] + degp_ref[1][:, 0:1] + 1.0
    dis = lax.rsqrt(deg)
    dis_ref[...] = jnp.broadcast_to(dis, (BM, 8))
    h1s_ref[...] = h_ref[...] * dis


def _k2_body(p_ref, h1s_ref, dis_ref, b1_ref, w2_ref, h2s_ref):
    dis = dis_ref[...][:, 0:1]
    a = (p_ref[0] + p_ref[1] + h1s_ref[...]) * dis + b1_ref[...]
    a = jnp.maximum(a, 0.0)
    h2 = jnp.dot(a, w2_ref[...], preferred_element_type=jnp.float32)
    h2s_ref[...] = h2 * dis


def _k3_body(q_ref, h2s_ref, dis_ref, b2_ref, out_ref):
    dis = dis_ref[...][:, 0:1]
    out_ref[...] = (q_ref[0] + q_ref[1] + h2s_ref[...]) * dis + b2_ref[...]


_row_spec = pl.BlockSpec((BM, D), lambda i: (i, 0))
_pair_spec = pl.BlockSpec((NC, BM, D), lambda i: (0, i, 0))
_deg_spec = pl.BlockSpec((NC, BM, D), lambda i: (0, i, 0))
_dis_spec = pl.BlockSpec((BM, 8), lambda i: (i, 0))
_w_spec = pl.BlockSpec((D, D), lambda i: (0, 0))
_b_spec = pl.BlockSpec((1, D), lambda i: (0, 0))

_k1a = pl.pallas_call(
    _k1a_body,
    grid=(NPAD // BM,),
    in_specs=[_row_spec, _w_spec],
    out_specs=_row_spec,
    out_shape=jax.ShapeDtypeStruct((NPAD, D), jnp.float32),
)

_k1 = pl.pallas_call(
    _k1_body,
    grid=(NPAD // BM,),
    in_specs=[_row_spec, _deg_spec],
    out_specs=[_row_spec, _dis_spec],
    out_shape=[jax.ShapeDtypeStruct((NPAD, D), jnp.float32),
               jax.ShapeDtypeStruct((NPAD, 8), jnp.float32)],
)

_k2 = pl.pallas_call(
    _k2_body,
    grid=(NPAD // BM,),
    in_specs=[_pair_spec, _row_spec, _dis_spec, _b_spec, _w_spec],
    out_specs=_row_spec,
    out_shape=jax.ShapeDtypeStruct((NPAD, D), jnp.float32),
)

_k3 = pl.pallas_call(
    _k3_body,
    grid=(NPAD // BM,),
    in_specs=[_pair_spec, _row_spec, _dis_spec, _b_spec],
    out_specs=_row_spec,
    out_shape=jax.ShapeDtypeStruct((NPAD, D), jnp.float32),
)


def kernel(x, edge_index, W1, b1, W2, b2):
    ei = edge_index.astype(jnp.int32)
    npad_extra = NPAD - N_NODES
    pad_n = NCHUNK * CHUNK - N_EDGES
    pad_lane = jnp.arange(pad_n, dtype=jnp.int32) % npad_extra
    src = jnp.concatenate([ei[0], N_NODES + pad_lane]).reshape(NCHUNK, CHUNK)
    dstf = jnp.concatenate([ei[1], N_NODES + pad_lane])
    dst = dstf.reshape(NCHUNK, CHUNK)

    xp = jnp.pad(x, ((0, npad_extra), (0, 0)))
    ones_deg = jnp.ones((CHUNK, D), jnp.float32)
    zeros_row = jnp.zeros((TROWS, D), jnp.float32)
    b1r = b1.reshape(1, D)
    b2r = b2.reshape(1, D)

    h1 = _k1a(xp, W1)
    degp = _deg_kernel(dst, ones_deg, zeros_row)
    h1s, dis = _k1(h1, degp)
    p = _mp_kernel(h1s, src, dst, zeros_row)
    h2s = _k2(p, h1s, dis, b1r, W2)
    q = _mp_kernel(h2s, src, dst, zeros_row)
    out = _k3(q, h2s, dis, b2r)
    return out[:N_NODES]


# speedup vs baseline: 1.0038x; 1.0038x over previous
import functools

import jax
import jax.numpy as jnp
from jax import lax
from jax.experimental import pallas as pl
from jax.experimental.pallas import tpu as pltpu
from jax.experimental.pallas import tpu_sc as plsc

N_NODES = 10000
N_EDGES = 320000
D = 128

NC, NS = 2, 16
NW = NC * NS
NPAD = 10240
CHUNK = 128
NCHUNK = 2560
CPW = NCHUNK // NW
TROWS = NPAD // NS

_mesh = plsc.VectorSubcoreMesh(
    core_axis_name="c", subcore_axis_name="s", num_cores=NC, num_subcores=NS
)


HALF = NPAD // 2
SEG = HALF // NS


@functools.partial(
    pl.kernel,
    mesh=_mesh,
    out_type=jax.ShapeDtypeStruct((NC, NPAD, D), jnp.float32),
    scratch_types=[
        pltpu.VMEM((CPW, CHUNK), jnp.int32),
        pltpu.VMEM((CHUNK, D), jnp.float32),
        pltpu.VMEM_SHARED((NPAD, D), jnp.float32),
        pltpu.SemaphoreType.DMA,
    ],
)
def _deg_kernel(dst_hbm, ones_hbm, zeros_hbm, out_hbm, dst_v, ones_v, acc, sem):
    c = lax.axis_index("c")
    s = lax.axis_index("s")
    w = s * NC + c
    pltpu.sync_copy(dst_hbm.at[pl.ds(w * CPW, CPW)], dst_v)
    pltpu.sync_copy(ones_hbm, ones_v)
    pltpu.sync_copy(zeros_hbm, acc.at[pl.ds(s * TROWS, TROWS)])
    plsc.subcore_barrier()

    for k in range(3):
        pltpu.async_copy(ones_v, acc.at[dst_v.at[k]], sem, add=True)

    def body(j, carry):
        @pl.when(j < CPW - 3)
        def _():
            pltpu.async_copy(ones_v, acc.at[dst_v.at[j + 3]], sem, add=True)

        pltpu.make_async_copy(ones_v, acc.at[dst_v.at[j]], sem).wait()
        return carry

    lax.fori_loop(0, CPW, body, 0)
    plsc.subcore_barrier()
    pltpu.sync_copy(
        acc.at[pl.ds(s * TROWS, TROWS)], out_hbm.at[c, pl.ds(s * TROWS, TROWS)]
    )


@functools.partial(
    pl.kernel,
    mesh=_mesh,
    out_type=jax.ShapeDtypeStruct((NC, NPAD, D), jnp.float32),
    scratch_types=[
        pltpu.VMEM(((CPW + 1) // 2, CHUNK), jnp.int32),
        pltpu.VMEM(((CPW + 1) // 2, CHUNK), jnp.int32),
        pltpu.VMEM((CHUNK, D), jnp.float32),
        pltpu.VMEM((CHUNK, D), jnp.float32),
        pltpu.VMEM_SHARED((NPAD, D), jnp.float32),
        pltpu.SemaphoreType.DMA,
        pltpu.SemaphoreType.DMA,
        pltpu.SemaphoreType.DMA,
        pltpu.SemaphoreType.DMA,
    ],
)
def _mp_kernel(h_hbm, src_hbm, dst_hbm, zeros_hbm, out_hbm,
               src_v, dst_v, g0, g1, acc, sg0, sg1, ss0, ss1):
    c = lax.axis_index("c")
    s = lax.axis_index("s")
    w = s * NC + c
    pltpu.sync_copy(zeros_hbm, acc.at[pl.ds(s * TROWS, TROWS)])
    plsc.subcore_barrier()

    half0 = (CPW + 1) // 2
    for h_i, half in enumerate((half0, CPW - half0)):
        base = w * CPW + h_i * half0
        pltpu.sync_copy(src_hbm.at[pl.ds(base, half)],
                        src_v.at[pl.ds(0, half)])
        pltpu.sync_copy(dst_hbm.at[pl.ds(base, half)],
                        dst_v.at[pl.ds(0, half)])
        npair = half // 2
        pltpu.async_copy(h_hbm.at[src_v.at[0]], g0, sg0)

        def body(g, carry):
            j0 = 2 * g
            j1 = j0 + 1
            pltpu.async_copy(h_hbm.at[src_v.at[j1]], g1, sg1)
            pltpu.make_async_copy(h_hbm.at[src_v.at[j0]], g0, sg0).wait()
            pltpu.async_copy(g0, acc.at[dst_v.at[j0]], ss0, add=True)
            pltpu.make_async_copy(h_hbm.at[src_v.at[j1]], g1, sg1).wait()
            pltpu.async_copy(g1, acc.at[dst_v.at[j1]], ss1, add=True)
            pltpu.make_async_copy(g0, acc.at[dst_v.at[j0]], ss0).wait()

            @pl.when(g < npair - 1)
            def _():
                pltpu.async_copy(h_hbm.at[src_v.at[j0 + 2]], g0, sg0)

            pltpu.make_async_copy(g1, acc.at[dst_v.at[j1]], ss1).wait()
            return carry

        lax.fori_loop(0, npair, body, 0)
        if half % 2 == 1:
            j = half - 1
            pltpu.async_copy(h_hbm.at[src_v.at[j]], g0, sg0)
            pltpu.make_async_copy(h_hbm.at[src_v.at[j]], g0, sg0).wait()
            pltpu.async_copy(g0, acc.at[dst_v.at[j]], ss0, add=True)
            pltpu.make_async_copy(g0, acc.at[dst_v.at[j]], ss0).wait()
    plsc.subcore_barrier()
    pltpu.sync_copy(
        acc.at[pl.ds(s * TROWS, TROWS)], out_hbm.at[c, pl.ds(s * TROWS, TROWS)]
    )


BM = 1280


def _k1_body(x_ref, w_ref, degp_ref, h1s_ref, dis_ref):
    deg = degp_ref[0][:, 0:1] + degp_ref[1][:, 0:1] + 1.0
    dis = lax.rsqrt(deg)
    dis_ref[...] = jnp.broadcast_to(dis, (BM, 8))
    h = jnp.dot(x_ref[...], w_ref[...], preferred_element_type=jnp.float32)
    h1s_ref[...] = h * dis


def _k2_body(p_ref, h1s_ref, dis_ref, b1_ref, w2_ref, h2s_ref):
    dis = dis_ref[...][:, 0:1]
    a = (p_ref[0] + p_ref[1] + h1s_ref[...]) * dis + b1_ref[...]
    a = jnp.maximum(a, 0.0)
    h2 = jnp.dot(a, w2_ref[...], preferred_element_type=jnp.float32)
    h2s_ref[...] = h2 * dis


def _k3_body(q_ref, h2s_ref, dis_ref, b2_ref, out_ref):
    dis = dis_ref[...][:, 0:1]
    out_ref[...] = (q_ref[0] + q_ref[1] + h2s_ref[...]) * dis + b2_ref[...]


_row_spec = pl.BlockSpec((BM, D), lambda i: (i, 0))
_pair_spec = pl.BlockSpec((NC, BM, D), lambda i: (0, i, 0))
_deg_spec = pl.BlockSpec((NC, BM, D), lambda i: (0, i, 0))
_dis_spec = pl.BlockSpec((BM, 8), lambda i: (i, 0))
_w_spec = pl.BlockSpec((D, D), lambda i: (0, 0))
_b_spec = pl.BlockSpec((1, D), lambda i: (0, 0))

_k1 = pl.pallas_call(
    _k1_body,
    grid=(NPAD // BM,),
    in_specs=[_row_spec, _w_spec, _deg_spec],
    out_specs=[_row_spec, _dis_spec],
    out_shape=[jax.ShapeDtypeStruct((NPAD, D), jnp.float32),
               jax.ShapeDtypeStruct((NPAD, 8), jnp.float32)],
)

_k2 = pl.pallas_call(
    _k2_body,
    grid=(NPAD // BM,),
    in_specs=[_pair_spec, _row_spec, _dis_spec, _b_spec, _w_spec],
    out_specs=_row_spec,
    out_shape=jax.ShapeDtypeStruct((NPAD, D), jnp.float32),
)

_k3 = pl.pallas_call(
    _k3_body,
    grid=(NPAD // BM,),
    in_specs=[_pair_spec, _row_spec, _dis_spec, _b_spec],
    out_specs=_row_spec,
    out_shape=jax.ShapeDtypeStruct((NPAD, D), jnp.float32),
)


def kernel(x, edge_index, W1, b1, W2, b2):
    ei = edge_index.astype(jnp.int32)
    npad_extra = NPAD - N_NODES
    pad_n = NCHUNK * CHUNK - N_EDGES
    pad_lane = jnp.arange(pad_n, dtype=jnp.int32) % npad_extra
    src = jnp.concatenate([ei[0], N_NODES + pad_lane]).reshape(NCHUNK, CHUNK)
    dstf = jnp.concatenate([ei[1], N_NODES + pad_lane])
    dst = dstf.reshape(NCHUNK, CHUNK)

    xp = jnp.pad(x, ((0, npad_extra), (0, 0)))
    ones_deg = jnp.ones((CHUNK, D), jnp.float32)
    zeros_row = jnp.zeros((TROWS, D), jnp.float32)
    b1r = b1.reshape(1, D)
    b2r = b2.reshape(1, D)

    degp = _deg_kernel(dst, ones_deg, zeros_row)
    h1s, dis = _k1(xp, W1, degp)
    p = _mp_kernel(h1s, src, dst, zeros_row)
    h2s = _k2(p, h1s, dis, b1r, W2)
    q = _mp_kernel(h2s, src, dst, zeros_row)
    out = _k3(q, h2s, dis, b2r)
    return out[:N_NODES]
